# Initial kernel scaffold; baseline (speedup 1.0000x reference)
import jax
import jax.numpy as jnp
from jax.experimental import pallas as pl


def kernel(features_batch, labels_batch):
    raise NotImplementedError("write your pallas kernel here")


# speedup vs baseline: 31.1886x; 31.1886x over previous
import functools

import jax
import jax.numpy as jnp
from jax import lax
from jax.experimental import pallas as pl
from jax.experimental.pallas import tpu as pltpu
from jax.experimental.pallas import tpu_sc as plsc

DD = 2.5
GAMMA = 0.005
NSEG = 16

B = 4
CH = 32
NPIX = 512 * 512

NC = 2
NS = 16
L = 16
NW = NC * NS
WPB = NW // B
PPW = NPIX // WPB
CHUNK = 1024
NCHUNK = PPW // CHUNK
NSTAT = CH + 2


def _sc_body(feat_hbm, lab_hbm, out_hbm, feat_v, lab_v, acc_v,
             semf0, semf1, seml0, seml1):
    cid = lax.axis_index("c")
    sid = lax.axis_index("s")
    wid = sid * NC + cid
    b = wid // WPB
    part = wid % WPB
    pbase = part * PPW

    semf = (semf0, semf1)
    seml = (seml0, seml1)

    for j in range(NSTAT):
        acc_v[pl.ds(j * L, L)] = jnp.zeros((L,), jnp.float32)

    def start(g, k):
        off = pbase + g * CHUNK
        pltpu.make_async_copy(
            feat_hbm.at[b, :, pl.ds(off, CHUNK)], feat_v.at[k], semf[k]).start()
        pltpu.make_async_copy(
            lab_hbm.at[b, pl.ds(off, CHUNK)], lab_v.at[k], seml[k]).start()

    def wait(k):
        pltpu.make_async_copy(
            feat_hbm.at[b, :, pl.ds(pbase, CHUNK)], feat_v.at[k], semf[k]).wait()
        pltpu.make_async_copy(
            lab_hbm.at[b, pl.ds(pbase, CHUNK)], lab_v.at[k], seml[k]).wait()

    def compute(k):
        def group(i, carry):
            base16 = i * L
            lab = lab_v[k, pl.ds(base16, L)]
            d2 = jnp.zeros((L,), jnp.float32)
            for c in range(CH):
                f = feat_v[k, c, pl.ds(base16, L)]
                plsc.addupdate_scatter(acc_v, [lab + (c * NSEG)], f)
                d2 = d2 + f * f
            plsc.addupdate_scatter(acc_v, [lab + (CH * NSEG)], d2)
            plsc.addupdate_scatter(acc_v, [lab + ((CH + 1) * NSEG)],
                                   jnp.ones((L,), jnp.float32))
            return carry
        lax.fori_loop(0, CHUNK // L, group, 0)

    start(0, 0)

    def outer(h, carry):
        for k in range(2):
            g = h * 2 + k

            @pl.when(g + 1 < NCHUNK)
            def _():
                start(g + 1, 1 - k)

            wait(k)
            compute(k)
        return carry

    lax.fori_loop(0, NCHUNK // 2, outer, 0)

    pltpu.sync_copy(acc_v, out_hbm.at[b, part])


@functools.partial(jax.jit, static_argnums=())
def _sc_stats(feat, lab):
    mesh = plsc.VectorSubcoreMesh(core_axis_name="c", subcore_axis_name="s")
    fn = functools.partial(
        pl.kernel,
        mesh=mesh,
        compiler_params=pltpu.CompilerParams(needs_layout_passes=False),
        out_type=jax.ShapeDtypeStruct((B, WPB, NSTAT * NSEG), jnp.float32),
        scratch_types=[
            pltpu.VMEM((2, CH, CHUNK), jnp.float32),
            pltpu.VMEM((2, CHUNK), jnp.int32),
            pltpu.VMEM((NSTAT * NSEG,), jnp.float32),
            pltpu.SemaphoreType.DMA,
            pltpu.SemaphoreType.DMA,
            pltpu.SemaphoreType.DMA,
            pltpu.SemaphoreType.DMA,
        ],
    )(_sc_body)
    return fn(feat, lab)


def _colb(row):
    ones = jnp.ones((1, NSEG), jnp.float32)
    return lax.dot_general(row, ones, (((0,), (0,)), ((), ())),
                           preferred_element_type=jnp.float32)


def _fin_kernel(p_ref, out_ref):
    bidx = pl.program_id(0)

    stats = p_ref[0, 0]
    for i in range(1, WPB):
        stats = stats + p_ref[0, i]

    sums = stats[0:CH, :]
    sumsq = stats[CH:CH + 1, :]
    counts = stats[CH + 1:CH + 2, :]

    safe = jnp.maximum(counts, 1.0)
    means = sums / safe
    msq = jnp.sum(means * means, axis=0, keepdims=True)
    seg_var = (sumsq - counts * msq) / safe

    seg_ids = lax.broadcasted_iota(jnp.int32, (1, NSEG), 1)
    valid = (seg_ids != 0) & (counts > 0.0)
    validf = valid.astype(jnp.float32)

    var_loss = jnp.sum(jnp.where(valid, seg_var, 0.0))
    ncl = jnp.sum(validf)

    g = lax.dot_general(means, means, (((0,), (0,)), ((), ())),
                        preferred_element_type=jnp.float32)
    sq = _colb(msq) + jnp.broadcast_to(msq, (NSEG, NSEG)) - 2.0 * g
    safe_sq = jnp.where(sq > 1e-12, sq, 1e-12)
    dist = jnp.sqrt(safe_sq)

    ii = lax.broadcasted_iota(jnp.int32, (NSEG, NSEG), 0)
    jj = lax.broadcasted_iota(jnp.int32, (NSEG, NSEG), 1)
    pair_mask = ((_colb(validf) > 0.5)
                 & (jnp.broadcast_to(validf, (NSEG, NSEG)) > 0.5)
                 & (ii < jj))
    hinge = jnp.where((dist < 2.0 * DD) & pair_mask, (2.0 * DD - dist) ** 2, 0.0)
    denom = jnp.maximum(ncl - 1.0, 1.0)
    dist_loss = jnp.where(ncl > 1.0, jnp.sum(hinge) / denom, 0.0)

    safe_msq = jnp.where(msq > 1e-12, msq, 1e-12)
    mnorm = jnp.sqrt(safe_msq)
    reg_loss = jnp.sum(jnp.where(valid, mnorm, 0.0))

    total = (var_loss + dist_loss + GAMMA * reg_loss) / jnp.maximum(ncl, 1.0)

    @pl.when(bidx == 0)
    def _():
        out_ref[0, 0] = 0.0

    out_ref[0, 0] += total / (B + 1.0)


def _finish(partials):
    return pl.pallas_call(
        _fin_kernel,
        grid=(B,),
        in_specs=[pl.BlockSpec((1, WPB, NSTAT, NSEG), lambda b: (b, 0, 0, 0))],
        out_specs=pl.BlockSpec(memory_space=pltpu.SMEM),
        out_shape=jax.ShapeDtypeStruct((1, 1), jnp.float32),
    )(partials)


def kernel(features_batch, labels_batch):
    feat = features_batch.reshape(B, CH, NPIX)
    lab = labels_batch.reshape(B, NPIX).astype(jnp.int32)
    partials = _sc_stats(feat, lab).reshape(B, WPB, NSTAT, NSEG)
    out = _finish(partials)
    return out[0, 0]


# speedup vs baseline: 47.3890x; 1.5194x over previous
import functools

import jax
import jax.numpy as jnp
from jax import lax
from jax.experimental import pallas as pl
from jax.experimental.pallas import tpu as pltpu
from jax.experimental.pallas import tpu_sc as plsc

DD = 2.5
GAMMA = 0.005
NSEG = 16

B = 4
CH = 32
NPIX = 512 * 512

NC = 2
NS = 16
L = 16
NW = NC * NS
WPB = NW // B
PPW = NPIX // WPB
CHUNK = 1024
NCHUNK = PPW // CHUNK
NSTAT = CH + 2


def _sc_body(feat_hbm, lab_hbm, out_hbm, feat_v, lab_v, acc_v,
             semf0, semf1, seml0, seml1):
    cid = lax.axis_index("c")
    sid = lax.axis_index("s")
    wid = sid * NC + cid
    b = wid // WPB
    part = wid % WPB
    pbase = part * PPW

    semf = (semf0, semf1)
    seml = (seml0, seml1)

    for j in range(NSTAT):
        acc_v[pl.ds(j * L, L)] = jnp.zeros((L,), jnp.float32)

    def start(g, k):
        off = pbase + g * CHUNK
        pltpu.make_async_copy(
            feat_hbm.at[b, :, pl.ds(off, CHUNK)], feat_v.at[k], semf[k]).start()
        pltpu.make_async_copy(
            lab_hbm.at[b, pl.ds(off, CHUNK)], lab_v.at[k], seml[k]).start()

    def wait(k):
        pltpu.make_async_copy(
            feat_hbm.at[b, :, pl.ds(pbase, CHUNK)], feat_v.at[k], semf[k]).wait()
        pltpu.make_async_copy(
            lab_hbm.at[b, pl.ds(pbase, CHUNK)], lab_v.at[k], seml[k]).wait()

    def compute(k):
        @plsc.parallel_loop(0, CHUNK // L, step=1, unroll=2)
        def group(i):
            base16 = i * L
            lab = lab_v[k, pl.ds(base16, L)]
            fs = [feat_v[k, c, pl.ds(base16, L)] for c in range(CH)]
            sq = [f * f for f in fs]
            while len(sq) > 1:
                sq = [sq[j] + sq[j + 1] for j in range(0, len(sq) - 1, 2)] + (
                    [sq[-1]] if len(sq) % 2 else [])
            for c in range(CH):
                plsc.addupdate_scatter(
                    acc_v.at[pl.ds(c * NSEG, NSEG)], [lab], fs[c])
            plsc.addupdate_scatter(
                acc_v.at[pl.ds(CH * NSEG, NSEG)], [lab], sq[0])
            plsc.addupdate_scatter(
                acc_v.at[pl.ds((CH + 1) * NSEG, NSEG)], [lab],
                jnp.ones((L,), jnp.float32))

    start(0, 0)

    def outer(h, carry):
        for k in range(2):
            g = h * 2 + k

            @pl.when(g + 1 < NCHUNK)
            def _():
                start(g + 1, 1 - k)

            wait(k)
            compute(k)
        return carry

    lax.fori_loop(0, NCHUNK // 2, outer, 0)

    pltpu.sync_copy(acc_v, out_hbm.at[b, part])


@functools.partial(jax.jit, static_argnums=())
def _sc_stats(feat, lab):
    mesh = plsc.VectorSubcoreMesh(core_axis_name="c", subcore_axis_name="s")
    fn = functools.partial(
        pl.kernel,
        mesh=mesh,
        compiler_params=pltpu.CompilerParams(needs_layout_passes=False),
        out_type=jax.ShapeDtypeStruct((B, WPB, NSTAT * NSEG), jnp.float32),
        scratch_types=[
            pltpu.VMEM((2, CH, CHUNK), jnp.float32),
            pltpu.VMEM((2, CHUNK), jnp.int32),
            pltpu.VMEM((NSTAT * NSEG,), jnp.float32),
            pltpu.SemaphoreType.DMA,
            pltpu.SemaphoreType.DMA,
            pltpu.SemaphoreType.DMA,
            pltpu.SemaphoreType.DMA,
        ],
    )(_sc_body)
    return fn(feat, lab)


def _colb(row):
    ones = jnp.ones((1, NSEG), jnp.float32)
    return lax.dot_general(row, ones, (((0,), (0,)), ((), ())),
                           preferred_element_type=jnp.float32)


def _fin_kernel(p_ref, out_ref):
    bidx = pl.program_id(0)

    stats = p_ref[0, 0]
    for i in range(1, WPB):
        stats = stats + p_ref[0, i]

    sums = stats[0:CH, :]
    sumsq = stats[CH:CH + 1, :]
    counts = stats[CH + 1:CH + 2, :]

    safe = jnp.maximum(counts, 1.0)
    means = sums / safe
    msq = jnp.sum(means * means, axis=0, keepdims=True)
    seg_var = (sumsq - counts * msq) / safe

    seg_ids = lax.broadcasted_iota(jnp.int32, (1, NSEG), 1)
    valid = (seg_ids != 0) & (counts > 0.0)
    validf = valid.astype(jnp.float32)

    var_loss = jnp.sum(jnp.where(valid, seg_var, 0.0))
    ncl = jnp.sum(validf)

    g = lax.dot_general(means, means, (((0,), (0,)), ((), ())),
                        preferred_element_type=jnp.float32)
    sq = _colb(msq) + jnp.broadcast_to(msq, (NSEG, NSEG)) - 2.0 * g
    safe_sq = jnp.where(sq > 1e-12, sq, 1e-12)
    dist = jnp.sqrt(safe_sq)

    ii = lax.broadcasted_iota(jnp.int32, (NSEG, NSEG), 0)
    jj = lax.broadcasted_iota(jnp.int32, (NSEG, NSEG), 1)
    pair_mask = ((_colb(validf) > 0.5)
                 & (jnp.broadcast_to(validf, (NSEG, NSEG)) > 0.5)
                 & (ii < jj))
    hinge = jnp.where((dist < 2.0 * DD) & pair_mask, (2.0 * DD - dist) ** 2, 0.0)
    denom = jnp.maximum(ncl - 1.0, 1.0)
    dist_loss = jnp.where(ncl > 1.0, jnp.sum(hinge) / denom, 0.0)

    safe_msq = jnp.where(msq > 1e-12, msq, 1e-12)
    mnorm = jnp.sqrt(safe_msq)
    reg_loss = jnp.sum(jnp.where(valid, mnorm, 0.0))

    total = (var_loss + dist_loss + GAMMA * reg_loss) / jnp.maximum(ncl, 1.0)

    @pl.when(bidx == 0)
    def _():
        out_ref[0, 0] = 0.0

    out_ref[0, 0] += total / (B + 1.0)


def _finish(partials):
    return pl.pallas_call(
        _fin_kernel,
        grid=(B,),
        in_specs=[pl.BlockSpec((1, WPB, NSTAT, NSEG), lambda b: (b, 0, 0, 0))],
        out_specs=pl.BlockSpec(memory_space=pltpu.SMEM),
        out_shape=jax.ShapeDtypeStruct((1, 1), jnp.float32),
    )(partials)


def kernel(features_batch, labels_batch):
    feat = features_batch.reshape(B, CH, NPIX)
    lab = labels_batch.reshape(B, NPIX).astype(jnp.int32)
    partials = _sc_stats(feat, lab).reshape(B, WPB, NSTAT, NSEG)
    out = _finish(partials)
    return out[0, 0]
